# Initial kernel scaffold; baseline (speedup 1.0000x reference)
import jax
import jax.numpy as jnp
from jax.experimental import pallas as pl


def kernel(x, reference_points, record_len, pairwise_t_matrix, lid, W_heat, b_heat, W_bev1, b_bev1, W_bev2, b_bev2):
    raise NotImplementedError("write your pallas kernel here")


# speedup vs baseline: 3.9208x; 3.9208x over previous
import functools

import jax
import jax.numpy as jnp
from jax import lax
from jax.experimental import pallas as pl
from jax.experimental.pallas import tpu as pltpu
from jax.experimental.pallas import tpu_sc as plsc

NQ = 2048
NCAV = 20
C = 256
B = 4
PER = 5
NN = 4
NPTS = NN * NQ
TOPK = 4096
KNN = 5
QT = 256
PC = (-140.8, -40.0, -3.0, 140.8, 40.0, 1.0)
BIG = 1e30
BIGI = 1 << 30


def _bfr(v):
    u = lax.bitcast_convert_type(v, jnp.uint32)
    u2 = u + jnp.uint32(0x7FFF) + ((u >> 16) & jnp.uint32(1))
    return lax.bitcast_convert_type(u2 & jnp.uint32(0xFFFF0000), jnp.float32)


def _stage1(x_ref, rp_ref, w_ref, t_ref, pen_ref, ego_ref, nbr_ref):
    xn = _bfr(x_ref[:, 0, 1:5, :])
    w = w_ref[...]
    s = jnp.sum(xn * w[None, :, :], axis=2)

    bi = lax.bitcast_convert_type(s, jnp.int32)
    key = bi ^ ((bi >> 31) & jnp.int32(0x7FFFFFFF))
    ku = lax.bitcast_convert_type(key, jnp.uint32) ^ jnp.uint32(0x80000000)

    def bit_body(i, t):
        cand = t | lax.shift_left(jnp.uint32(1), (31 - i).astype(jnp.uint32))
        cnt = jnp.sum((ku >= cand).astype(jnp.int32))
        return jnp.where(cnt >= TOPK, cand, t)

    thr = lax.fori_loop(0, 32, bit_body, jnp.uint32(0))
    pen_ref[0] = jnp.where(ku >= thr, 0.0, BIG).astype(jnp.float32)

    for i in range(PER):
        sx = _bfr(rp_ref[i, 0, :] * (PC[3] - PC[0]) + PC[0])
        sy = _bfr(rp_ref[i, 1, :] * (PC[4] - PC[1]) + PC[1])
        sz = _bfr(rp_ref[i, 2, :] * (PC[5] - PC[2]) + PC[2])
        for k in range(3):
            t0 = t_ref[0, 0, i * 12 + k * 4 + 0]
            t1 = t_ref[0, 0, i * 12 + k * 4 + 1]
            t2 = t_ref[0, 0, i * 12 + k * 4 + 2]
            t3 = t_ref[0, 0, i * 12 + k * 4 + 3]
            v = sx * t0 + sy * t1 + sz * t2 + t3
            if i == 0:
                ego_ref[0, k, :] = v
            else:
                nbr_ref[0, k, pl.ds((i - 1) * NQ, NQ)] = v


def _stage2(ego_ref, nbr_ref, pen_ref, out_ref):
    b = pl.program_id(0)
    qt = pl.program_id(1)
    e = ego_ref[...]
    pen = pen_ref[...][0]
    ex = e[0, :, 0:1]
    ey = e[0, :, 1:2]
    ez = e[0, :, 2:3]
    slabs = []
    for n in range(NN):
        px = nbr_ref[0, 0, pl.ds(n * NQ, NQ)][None, :]
        py = nbr_ref[0, 1, pl.ds(n * NQ, NQ)][None, :]
        pz = nbr_ref[0, 2, pl.ds(n * NQ, NQ)][None, :]
        dx = ex - px
        dy = ey - py
        dz = ez - pz
        slabs.append(dx * dx + dy * dy + dz * dz + pen[:, n * NQ:(n + 1) * NQ])
    d2 = jnp.concatenate(slabs, axis=1)
    iota = lax.broadcasted_iota(jnp.int32, (QT, NPTS), 1)
    cols = []
    for _ in range(KNN):
        m = jnp.min(d2, axis=1, keepdims=True)
        jsel = jnp.min(jnp.where(d2 == m, iota, BIGI), axis=1, keepdims=True)
        d2 = jnp.where(iota == jsel, BIG, d2)
        cols.append((jsel & (NQ - 1)) * NCAV + (5 * b + 1) + (jsel >> 11))
    qv = qt * QT + lax.broadcasted_iota(jnp.int32, (QT, 1), 0)
    egorow = qv * NCAV + 5 * b
    cols += [egorow, egorow, egorow]
    out_ref[0] = jnp.concatenate(cols, axis=1)


def _knn_indices(x, reference_points, pairwise_t_matrix, W_heat):
    rpT = jnp.transpose(reference_points, (0, 2, 1))
    Tp = _bfr(pairwise_t_matrix[:, :, 0, :3, :].astype(jnp.float32)).reshape(B, 1, 60)
    w2 = _bfr(W_heat.reshape(1, C))

    pen, ego, nbr = pl.pallas_call(
        _stage1,
        grid=(B,),
        in_specs=[
            pl.BlockSpec((NQ, 1, PER, C), lambda b: (0, b, 0, 0)),
            pl.BlockSpec((PER, 3, NQ), lambda b: (b, 0, 0)),
            pl.BlockSpec((1, C), lambda b: (0, 0)),
            pl.BlockSpec((1, 1, 60), lambda b: (b, 0, 0),
                         memory_space=pltpu.SMEM),
        ],
        out_specs=[
            pl.BlockSpec((1, NQ, NN), lambda b: (b, 0, 0)),
            pl.BlockSpec((1, 3, NQ), lambda b: (b, 0, 0)),
            pl.BlockSpec((1, 3, NPTS), lambda b: (b, 0, 0)),
        ],
        out_shape=[
            jax.ShapeDtypeStruct((B, NQ, NN), jnp.float32),
            jax.ShapeDtypeStruct((B, 3, NQ), jnp.float32),
            jax.ShapeDtypeStruct((B, 3, NPTS), jnp.float32),
        ],
    )(x.reshape(NQ, B, PER, C), rpT, w2, Tp)

    pen2 = jnp.transpose(pen, (0, 2, 1)).reshape(B, 1, NPTS)
    egoT = jnp.transpose(ego, (0, 2, 1))

    idx = pl.pallas_call(
        _stage2,
        grid=(B, NQ // QT),
        in_specs=[
            pl.BlockSpec((1, QT, 3), lambda b, q: (b, q, 0)),
            pl.BlockSpec((1, 3, NPTS), lambda b, q: (b, 0, 0)),
            pl.BlockSpec((1, 1, NPTS), lambda b, q: (b, 0, 0)),
        ],
        out_specs=pl.BlockSpec((1, QT, 8), lambda b, q: (b, q, 0)),
        out_shape=jax.ShapeDtypeStruct((B, NQ, 8), jnp.int32),
    )(egoT, nbr, pen2)
    return idx.reshape(-1)


def _gather_max(x2d, idx_flat):
    nw = 32
    qw = (B * NQ) // nw
    nchunk = qw // 16

    @functools.partial(
        pl.kernel,
        out_type=jax.ShapeDtypeStruct((B * NQ, C), jnp.float32),
        mesh=plsc.VectorSubcoreMesh(core_axis_name="c", subcore_axis_name="s"),
        scratch_types=[
            pltpu.VMEM((128,), jnp.int32),
            pltpu.VMEM((128, C), jnp.float32),
            pltpu.VMEM((16, C), jnp.float32),
            pltpu.SemaphoreType.DMA,
        ],
    )
    def sc_kernel(x_hbm, idx_hbm, out_hbm, idxbuf, rows, outbuf, sem):
        wid = lax.axis_index("s") * 2 + lax.axis_index("c")

        def chunk_body(t, carry):
            g0 = wid * qw + t * 16
            pltpu.sync_copy(idx_hbm.at[pl.ds(g0 * 8, 128)], idxbuf)
            pltpu.async_copy(x_hbm.at[idxbuf], rows, sem).wait()

            def lane_body(cc, carry2):
                sl = pl.ds(cc * 16, 16)
                for qi in range(16):
                    r0 = 8 * qi
                    m01 = jnp.maximum(rows[r0 + 0, sl], rows[r0 + 1, sl])
                    m23 = jnp.maximum(rows[r0 + 2, sl], rows[r0 + 3, sl])
                    m = jnp.maximum(jnp.maximum(m01, m23), rows[r0 + 4, sl])
                    outbuf[qi, sl] = m + rows[r0 + 5, sl]
                return carry2

            lax.fori_loop(0, C // 16, lane_body, 0)
            pltpu.sync_copy(outbuf, out_hbm.at[pl.ds(g0, 16)])
            return carry

        lax.fori_loop(0, nchunk, chunk_body, 0)

    return sc_kernel(x2d, idx_flat)


def kernel(x, reference_points, record_len, pairwise_t_matrix, lid,
           W_heat, b_heat, W_bev1, b_bev1, W_bev2, b_bev2):
    idx_flat = _knn_indices(x, reference_points, pairwise_t_matrix, W_heat)
    x2d = x.reshape(NQ * NCAV, C)
    out_ego = _gather_max(x2d, idx_flat)
    new_ego = jnp.transpose(out_ego.reshape(B, NQ, C), (1, 0, 2))
    return x.at[:, jnp.array([0, 5, 10, 15]), :].set(new_ego)


# speedup vs baseline: 3.9254x; 1.0012x over previous
import functools

import jax
import jax.numpy as jnp
from jax import lax
from jax.experimental import pallas as pl
from jax.experimental.pallas import tpu as pltpu
from jax.experimental.pallas import tpu_sc as plsc

NQ = 2048
NCAV = 20
C = 256
B = 4
PER = 5
NN = 4
NPTS = NN * NQ
TOPK = 4096
KNN = 5
QT = 256
PC = (-140.8, -40.0, -3.0, 140.8, 40.0, 1.0)
BIG = 1e30
BIGI = 1 << 30


def _bfr(v):
    u = lax.bitcast_convert_type(v, jnp.uint32)
    u2 = u + jnp.uint32(0x7FFF) + ((u >> 16) & jnp.uint32(1))
    return lax.bitcast_convert_type(u2 & jnp.uint32(0xFFFF0000), jnp.float32)


def _stage1(x_ref, rp_ref, w_ref, t_ref, pen_ref, ego_ref, nbr_ref):
    xn = _bfr(x_ref[:, 0, 1:5, :])
    w = w_ref[...]
    s = jnp.sum(xn * w[None, :, :], axis=2)

    bi = lax.bitcast_convert_type(s, jnp.int32)
    key = bi ^ ((bi >> 31) & jnp.int32(0x7FFFFFFF))
    ku = lax.bitcast_convert_type(key, jnp.uint32) ^ jnp.uint32(0x80000000)

    def bit_body(i, t):
        cand = t | lax.shift_left(jnp.uint32(1), (31 - i).astype(jnp.uint32))
        cnt = jnp.sum((ku >= cand).astype(jnp.int32))
        return jnp.where(cnt >= TOPK, cand, t)

    thr = lax.fori_loop(0, 32, bit_body, jnp.uint32(0))
    pen_ref[0] = jnp.where(ku >= thr, 0.0, BIG).astype(jnp.float32)

    for i in range(PER):
        sx = _bfr(rp_ref[i, 0, :] * (PC[3] - PC[0]) + PC[0])
        sy = _bfr(rp_ref[i, 1, :] * (PC[4] - PC[1]) + PC[1])
        sz = _bfr(rp_ref[i, 2, :] * (PC[5] - PC[2]) + PC[2])
        for k in range(3):
            t0 = t_ref[0, 0, i * 12 + k * 4 + 0]
            t1 = t_ref[0, 0, i * 12 + k * 4 + 1]
            t2 = t_ref[0, 0, i * 12 + k * 4 + 2]
            t3 = t_ref[0, 0, i * 12 + k * 4 + 3]
            v = sx * t0 + sy * t1 + sz * t2 + t3
            if i == 0:
                ego_ref[0, k, :] = v
            else:
                nbr_ref[0, k, pl.ds((i - 1) * NQ, NQ)] = v


def _stage2(ego_ref, nbr_ref, pen_ref, out_ref):
    b = pl.program_id(0)
    qt = pl.program_id(1)
    e = ego_ref[...]
    pen = pen_ref[...][0]
    ex = e[0, :, 0:1]
    ey = e[0, :, 1:2]
    ez = e[0, :, 2:3]
    slabs = []
    for n in range(NN):
        px = nbr_ref[0, 0, pl.ds(n * NQ, NQ)][None, :]
        py = nbr_ref[0, 1, pl.ds(n * NQ, NQ)][None, :]
        pz = nbr_ref[0, 2, pl.ds(n * NQ, NQ)][None, :]
        dx = ex - px
        dy = ey - py
        dz = ez - pz
        slabs.append(dx * dx + dy * dy + dz * dz + pen[:, n * NQ:(n + 1) * NQ])
    d2 = jnp.concatenate(slabs, axis=1)
    iota = lax.broadcasted_iota(jnp.int32, (QT, NPTS), 1)
    cols = []
    for r in range(KNN):
        m = jnp.min(d2, axis=1, keepdims=True)
        jsel = jnp.min(jnp.where(d2 == m, iota, BIGI), axis=1, keepdims=True)
        if r < KNN - 1:
            d2 = jnp.where(iota == jsel, BIG, d2)
        cols.append((jsel & (NQ - 1)) * NCAV + (5 * b + 1) + (jsel >> 11))
    qv = qt * QT + lax.broadcasted_iota(jnp.int32, (QT, 1), 0)
    egorow = qv * NCAV + 5 * b
    cols += [egorow, egorow, egorow]
    out_ref[0] = jnp.concatenate(cols, axis=1)


def _knn_indices(x, reference_points, pairwise_t_matrix, W_heat):
    rpT = jnp.transpose(reference_points, (0, 2, 1))
    Tp = _bfr(pairwise_t_matrix[:, :, 0, :3, :].astype(jnp.float32)).reshape(B, 1, 60)
    w2 = _bfr(W_heat.reshape(1, C))

    pen, ego, nbr = pl.pallas_call(
        _stage1,
        grid=(B,),
        in_specs=[
            pl.BlockSpec((NQ, 1, PER, C), lambda b: (0, b, 0, 0)),
            pl.BlockSpec((PER, 3, NQ), lambda b: (b, 0, 0)),
            pl.BlockSpec((1, C), lambda b: (0, 0)),
            pl.BlockSpec((1, 1, 60), lambda b: (b, 0, 0),
                         memory_space=pltpu.SMEM),
        ],
        out_specs=[
            pl.BlockSpec((1, NQ, NN), lambda b: (b, 0, 0)),
            pl.BlockSpec((1, 3, NQ), lambda b: (b, 0, 0)),
            pl.BlockSpec((1, 3, NPTS), lambda b: (b, 0, 0)),
        ],
        out_shape=[
            jax.ShapeDtypeStruct((B, NQ, NN), jnp.float32),
            jax.ShapeDtypeStruct((B, 3, NQ), jnp.float32),
            jax.ShapeDtypeStruct((B, 3, NPTS), jnp.float32),
        ],
    )(x.reshape(NQ, B, PER, C), rpT, w2, Tp)

    pen2 = jnp.transpose(pen, (0, 2, 1)).reshape(B, 1, NPTS)
    egoT = jnp.transpose(ego, (0, 2, 1))

    idx = pl.pallas_call(
        _stage2,
        grid=(B, NQ // QT),
        in_specs=[
            pl.BlockSpec((1, QT, 3), lambda b, q: (b, q, 0)),
            pl.BlockSpec((1, 3, NPTS), lambda b, q: (b, 0, 0)),
            pl.BlockSpec((1, 1, NPTS), lambda b, q: (b, 0, 0)),
        ],
        out_specs=pl.BlockSpec((1, QT, 8), lambda b, q: (b, q, 0)),
        out_shape=jax.ShapeDtypeStruct((B, NQ, 8), jnp.int32),
    )(egoT, nbr, pen2)
    return idx.reshape(-1)


def _gather_max(x2d, idx_flat):
    nw = 32
    qw = (B * NQ) // nw
    nchunk = qw // 16

    @functools.partial(
        pl.kernel,
        out_type=jax.ShapeDtypeStruct((B * NQ, C), jnp.float32),
        mesh=plsc.VectorSubcoreMesh(core_axis_name="c", subcore_axis_name="s"),
        scratch_types=[
            pltpu.VMEM((128,), jnp.int32),
            pltpu.VMEM((128, C), jnp.float32),
            pltpu.VMEM((16, C), jnp.float32),
            pltpu.SemaphoreType.DMA,
        ],
    )
    def sc_kernel(x_hbm, idx_hbm, out_hbm, idxbuf, rows, outbuf, sem):
        wid = lax.axis_index("s") * 2 + lax.axis_index("c")

        def chunk_body(t, carry):
            g0 = wid * qw + t * 16
            pltpu.sync_copy(idx_hbm.at[pl.ds(g0 * 8, 128)], idxbuf)
            pltpu.async_copy(x_hbm.at[idxbuf], rows, sem).wait()

            def lane_body(cc, carry2):
                sl = pl.ds(cc * 16, 16)
                for qi in range(16):
                    r0 = 8 * qi
                    m01 = jnp.maximum(rows[r0 + 0, sl], rows[r0 + 1, sl])
                    m23 = jnp.maximum(rows[r0 + 2, sl], rows[r0 + 3, sl])
                    m = jnp.maximum(jnp.maximum(m01, m23), rows[r0 + 4, sl])
                    outbuf[qi, sl] = m + rows[r0 + 5, sl]
                return carry2

            lax.fori_loop(0, C // 16, lane_body, 0)
            pltpu.sync_copy(outbuf, out_hbm.at[pl.ds(g0, 16)])
            return carry

        lax.fori_loop(0, nchunk, chunk_body, 0)

    return sc_kernel(x2d, idx_flat)


def kernel(x, reference_points, record_len, pairwise_t_matrix, lid,
           W_heat, b_heat, W_bev1, b_bev1, W_bev2, b_bev2):
    idx_flat = _knn_indices(x, reference_points, pairwise_t_matrix, W_heat)
    x2d = x.reshape(NQ * NCAV, C)
    out_ego = _gather_max(x2d, idx_flat)
    new_ego = jnp.transpose(out_ego.reshape(B, NQ, C), (1, 0, 2))
    return x.at[:, jnp.array([0, 5, 10, 15]), :].set(new_ego)
